# baseline (speedup 1.0000x reference)
import math

import jax
import jax.numpy as jnp
from jax import lax
from jax.experimental import pallas as pl
from jax.experimental.pallas import tpu as pltpu
from jax.experimental.pallas import tpu_sc as plsc

_SNR_W = 0.5
_ENERGY_W = 0.3
_RECENCY_C = 0.2 * math.exp(-0.1)

_N_TOTAL = 4 * 32 * 128 * 128
_NC = 2
_NS = 16
_NW = _NC * _NS
_L = 16
_CHUNK = _N_TOTAL // _NW
_NV = _CHUNK // _L

_mesh = plsc.VectorSubcoreMesh(core_axis_name="c", subcore_axis_name="s")


def _worker_id():
    return lax.axis_index("s") * _NC + lax.axis_index("c")


def _reduce_body(x_hbm, part_hbm, buf, part):
    wid = _worker_id()
    pltpu.sync_copy(x_hbm.at[pl.ds(wid * _CHUNK, _CHUNK)], buf)

    zero = jnp.zeros((_L,), jnp.float32)

    def body(i, carry):
        s, ss, m = carry
        v = buf[pl.ds(i * _L, _L)]
        return s + v, ss + v * v, jnp.maximum(m, jnp.abs(v))

    s, ss, m = lax.fori_loop(0, _NV, body, (zero, zero, zero))
    part[0] = s
    part[1] = ss
    part[2] = m
    pltpu.sync_copy(part, part_hbm.at[wid])


def _lane_all_reduce(v, op):
    lanes = lax.iota(jnp.int32, _L)
    dnums = lax.GatherDimensionNumbers(
        offset_dims=(), collapsed_slice_dims=(0,), start_index_map=(0,)
    )
    for k in (8, 4, 2, 1):
        idx = lax.bitwise_xor(lanes, jnp.int32(k))
        perm = lax.gather(
            v,
            idx.reshape(_L, 1),
            dnums,
            (1,),
            mode=lax.GatherScatterMode.PROMISE_IN_BOUNDS,
        )
        v = op(v, perm)
    return v


def _sqrt_vec(a):
    ai = lax.bitcast_convert_type(a, jnp.int32)
    yi = lax.shift_right_logical(ai, 1) + jnp.int32(0x1FBD1DF5)
    y = lax.bitcast_convert_type(yi, jnp.float32)
    for _ in range(4):
        y = 0.5 * (y + a / y)
    return y


def _score_body(x_hbm, part_hbm, out_hbm, buf, parts):
    wid = _worker_id()
    pltpu.sync_copy(x_hbm.at[pl.ds(wid * _CHUNK, _CHUNK)], buf)
    pltpu.sync_copy(part_hbm, parts)

    zero = jnp.zeros((_L,), jnp.float32)
    s_v, ss_v, m_v = zero, zero, zero
    for w in range(_NW):
        s_v = s_v + parts[w, 0]
        ss_v = ss_v + parts[w, 1]
        m_v = jnp.maximum(m_v, parts[w, 2])

    n = jnp.float32(_N_TOTAL)
    total_s = _lane_all_reduce(s_v, jnp.add)
    total_ss = _lane_all_reduce(ss_v, jnp.add)
    max_abs = _lane_all_reduce(m_v, jnp.maximum)
    var = (total_ss - total_s * total_s / n) / (n - 1.0)
    sigma = _sqrt_vec(var)
    inv_sig = 1.0 / (sigma + 1e-6)
    k_e = _ENERGY_W / (max_abs * max_abs + 1e-6)

    def body(i, _):
        v = buf[pl.ds(i * _L, _L)]
        e = jnp.exp(2.0 - jnp.abs(v) * inv_sig)
        buf[pl.ds(i * _L, _L)] = _SNR_W / (1.0 + e) + k_e * (v * v) + _RECENCY_C
        return 0

    lax.fori_loop(0, _NV, body, 0)
    pltpu.sync_copy(buf, out_hbm.at[pl.ds(wid * _CHUNK, _CHUNK)])


def kernel(weights):
    x = weights.reshape(_N_TOTAL)
    partials = pl.kernel(
        _reduce_body,
        mesh=_mesh,
        out_type=jax.ShapeDtypeStruct((_NW, 3, _L), jnp.float32),
        scratch_types=[
            pltpu.VMEM((_CHUNK,), jnp.float32),
            pltpu.VMEM((3, _L), jnp.float32),
        ],
    )(x)
    out = pl.kernel(
        _score_body,
        mesh=_mesh,
        out_type=jax.ShapeDtypeStruct((_N_TOTAL,), jnp.float32),
        scratch_types=[
            pltpu.VMEM((_CHUNK,), jnp.float32),
            pltpu.VMEM((_NW, 3, _L), jnp.float32),
        ],
    )(x, partials)
    return out.reshape(weights.shape)


# speedup vs baseline: 2.7930x; 2.7930x over previous
import math

import jax
import jax.numpy as jnp
from jax import lax
from jax.experimental import pallas as pl
from jax.experimental.pallas import tpu as pltpu
from jax.experimental.pallas import tpu_sc as plsc

_SNR_W = 0.5
_ENERGY_W = 0.3
_RECENCY_C = 0.2 * math.exp(-0.1)

_N_TOTAL = 4 * 32 * 128 * 128
_NC = 2
_NS = 16
_NW = _NC * _NS
_L = 16
_CHUNK = _N_TOTAL // _NW
_NV = _CHUNK // _L

_mesh = plsc.VectorSubcoreMesh(core_axis_name="c", subcore_axis_name="s")


def _worker_id():
    return lax.axis_index("s") * _NC + lax.axis_index("c")


def _reduce_body(x_hbm, part_hbm, buf, part):
    wid = _worker_id()
    pltpu.sync_copy(x_hbm.at[pl.ds(wid * _CHUNK, _CHUNK)], buf)

    zero = jnp.zeros((_L,), jnp.float32)

    def body(i, carry):
        s, ss, m = carry
        v = buf[pl.ds(i * _L, _L)]
        return s + v, ss + v * v, jnp.maximum(m, jnp.abs(v))

    s, ss, m = lax.fori_loop(0, _NV, body, (zero, zero, zero), unroll=8)
    part[0] = s
    part[1] = ss
    part[2] = m
    pltpu.sync_copy(part, part_hbm.at[wid])


def _lane_all_reduce(v, op):
    lanes = lax.iota(jnp.int32, _L)
    dnums = lax.GatherDimensionNumbers(
        offset_dims=(), collapsed_slice_dims=(0,), start_index_map=(0,)
    )
    for k in (8, 4, 2, 1):
        idx = lax.bitwise_xor(lanes, jnp.int32(k))
        perm = lax.gather(
            v,
            idx.reshape(_L, 1),
            dnums,
            (1,),
            mode=lax.GatherScatterMode.PROMISE_IN_BOUNDS,
        )
        v = op(v, perm)
    return v


def _sqrt_vec(a):
    ai = lax.bitcast_convert_type(a, jnp.int32)
    yi = lax.shift_right_logical(ai, 1) + jnp.int32(0x1FBD1DF5)
    y = lax.bitcast_convert_type(yi, jnp.float32)
    for _ in range(4):
        y = 0.5 * (y + a / y)
    return y


def _score_body(x_hbm, part_hbm, out_hbm, buf, parts):
    wid = _worker_id()
    pltpu.sync_copy(x_hbm.at[pl.ds(wid * _CHUNK, _CHUNK)], buf)
    pltpu.sync_copy(part_hbm, parts)

    zero = jnp.zeros((_L,), jnp.float32)
    s_v, ss_v, m_v = zero, zero, zero
    for w in range(_NW):
        s_v = s_v + parts[w, 0]
        ss_v = ss_v + parts[w, 1]
        m_v = jnp.maximum(m_v, parts[w, 2])

    n = jnp.float32(_N_TOTAL)
    total_s = _lane_all_reduce(s_v, jnp.add)
    total_ss = _lane_all_reduce(ss_v, jnp.add)
    max_abs = _lane_all_reduce(m_v, jnp.maximum)
    var = (total_ss - total_s * total_s / n) / (n - 1.0)
    sigma = _sqrt_vec(var)
    inv_sig = 1.0 / (sigma + 1e-6)
    k_e = _ENERGY_W / (max_abs * max_abs + 1e-6)

    def body(i, _):
        v = buf[pl.ds(i * _L, _L)]
        e = jnp.exp(2.0 - jnp.abs(v) * inv_sig)
        buf[pl.ds(i * _L, _L)] = _SNR_W / (1.0 + e) + k_e * (v * v) + _RECENCY_C
        return 0

    lax.fori_loop(0, _NV, body, 0, unroll=8)
    pltpu.sync_copy(buf, out_hbm.at[pl.ds(wid * _CHUNK, _CHUNK)])


def kernel(weights):
    x = weights.reshape(_N_TOTAL)
    partials = pl.kernel(
        _reduce_body,
        mesh=_mesh,
        out_type=jax.ShapeDtypeStruct((_NW, 3, _L), jnp.float32),
        scratch_types=[
            pltpu.VMEM((_CHUNK,), jnp.float32),
            pltpu.VMEM((3, _L), jnp.float32),
        ],
    )(x)
    out = pl.kernel(
        _score_body,
        mesh=_mesh,
        out_type=jax.ShapeDtypeStruct((_N_TOTAL,), jnp.float32),
        scratch_types=[
            pltpu.VMEM((_CHUNK,), jnp.float32),
            pltpu.VMEM((_NW, 3, _L), jnp.float32),
        ],
    )(x, partials)
    return out.reshape(weights.shape)


# speedup vs baseline: 2.9738x; 1.0647x over previous
import math

import jax
import jax.numpy as jnp
from jax import lax
from jax.experimental import pallas as pl
from jax.experimental.pallas import tpu as pltpu
from jax.experimental.pallas import tpu_sc as plsc

_SNR_W = 0.5
_ENERGY_W = 0.3
_RECENCY_C = 0.2 * math.exp(-0.1)

_N_TOTAL = 4 * 32 * 128 * 128
_NC = 2
_NS = 16
_NW = _NC * _NS
_L = 16
_CHUNK = _N_TOTAL // _NW
_NV = _CHUNK // _L

_mesh = plsc.VectorSubcoreMesh(core_axis_name="c", subcore_axis_name="s")


def _worker_id():
    return lax.axis_index("s") * _NC + lax.axis_index("c")


def _reduce_body(x_hbm, part_hbm, buf, part):
    wid = _worker_id()
    pltpu.sync_copy(x_hbm.at[pl.ds(wid * _CHUNK, _CHUNK)], buf)

    zero = jnp.zeros((_L,), jnp.float32)

    def body(i, carry):
        s, ss, m = carry
        v = buf[pl.ds(i * _L, _L)]
        return s + v, ss + v * v, jnp.maximum(m, jnp.abs(v))

    s, ss, m = lax.fori_loop(0, _NV, body, (zero, zero, zero), unroll=16)
    part[0] = s
    part[1] = ss
    part[2] = m
    pltpu.sync_copy(part, part_hbm.at[wid])


def _lane_all_reduce(v, op):
    lanes = lax.iota(jnp.int32, _L)
    dnums = lax.GatherDimensionNumbers(
        offset_dims=(), collapsed_slice_dims=(0,), start_index_map=(0,)
    )
    for k in (8, 4, 2, 1):
        idx = lax.bitwise_xor(lanes, jnp.int32(k))
        perm = lax.gather(
            v,
            idx.reshape(_L, 1),
            dnums,
            (1,),
            mode=lax.GatherScatterMode.PROMISE_IN_BOUNDS,
        )
        v = op(v, perm)
    return v


def _sqrt_vec(a):
    ai = lax.bitcast_convert_type(a, jnp.int32)
    yi = lax.shift_right_logical(ai, 1) + jnp.int32(0x1FBD1DF5)
    y = lax.bitcast_convert_type(yi, jnp.float32)
    for _ in range(4):
        y = 0.5 * (y + a / y)
    return y


def _score_body(x_hbm, part_hbm, out_hbm, buf, parts):
    wid = _worker_id()
    pltpu.sync_copy(x_hbm.at[pl.ds(wid * _CHUNK, _CHUNK)], buf)
    pltpu.sync_copy(part_hbm, parts)

    zero = jnp.zeros((_L,), jnp.float32)
    s_v, ss_v, m_v = zero, zero, zero
    for w in range(_NW):
        s_v = s_v + parts[w, 0]
        ss_v = ss_v + parts[w, 1]
        m_v = jnp.maximum(m_v, parts[w, 2])

    n = jnp.float32(_N_TOTAL)
    total_s = _lane_all_reduce(s_v, jnp.add)
    total_ss = _lane_all_reduce(ss_v, jnp.add)
    max_abs = _lane_all_reduce(m_v, jnp.maximum)
    var = (total_ss - total_s * total_s / n) / (n - 1.0)
    sigma = _sqrt_vec(var)
    inv_sig = 1.0 / (sigma + 1e-6)
    k_e = _ENERGY_W / (max_abs * max_abs + 1e-6)

    def body(i, _):
        v = buf[pl.ds(i * _L, _L)]
        e = jnp.exp(2.0 - jnp.abs(v) * inv_sig)
        buf[pl.ds(i * _L, _L)] = _SNR_W / (1.0 + e) + k_e * (v * v) + _RECENCY_C
        return 0

    lax.fori_loop(0, _NV, body, 0, unroll=16)
    pltpu.sync_copy(buf, out_hbm.at[pl.ds(wid * _CHUNK, _CHUNK)])


def kernel(weights):
    x = weights.reshape(_N_TOTAL)
    partials = pl.kernel(
        _reduce_body,
        mesh=_mesh,
        out_type=jax.ShapeDtypeStruct((_NW, 3, _L), jnp.float32),
        scratch_types=[
            pltpu.VMEM((_CHUNK,), jnp.float32),
            pltpu.VMEM((3, _L), jnp.float32),
        ],
    )(x)
    out = pl.kernel(
        _score_body,
        mesh=_mesh,
        out_type=jax.ShapeDtypeStruct((_N_TOTAL,), jnp.float32),
        scratch_types=[
            pltpu.VMEM((_CHUNK,), jnp.float32),
            pltpu.VMEM((_NW, 3, _L), jnp.float32),
        ],
    )(x, partials)
    return out.reshape(weights.shape)


# speedup vs baseline: 16.9843x; 5.7113x over previous
import math

import jax
import jax.numpy as jnp
from jax.experimental import pallas as pl
from jax.experimental.pallas import tpu as pltpu

_SNR_W = 0.5
_ENERGY_W = 0.3
_RECENCY_C = 0.2 * math.exp(-0.1)

_N_TOTAL = 4 * 32 * 128 * 128
_ROWS = _N_TOTAL // 128
_CH = 2048
_NCHUNK = _ROWS // _CH


def _fused_kernel(x_hbm, o_hbm, x_vmem, sem_in, sem_out):
    for i in range(_NCHUNK):
        pltpu.make_async_copy(
            x_hbm.at[pl.ds(i * _CH, _CH)],
            x_vmem.at[pl.ds(i * _CH, _CH)],
            sem_in.at[i],
        ).start()

    def p1(g, carry):
        s, ss, m = carry
        pltpu.make_async_copy(
            x_hbm.at[pl.ds(g * _CH, _CH)],
            x_vmem.at[pl.ds(g * _CH, _CH)],
            sem_in.at[g],
        ).wait()
        x = x_vmem[pl.ds(g * _CH, _CH), :].reshape(_CH // 8, 8, 128)
        s = s + jnp.sum(x, axis=0)
        ss = ss + jnp.sum(x * x, axis=0)
        m = jnp.maximum(m, jnp.max(jnp.abs(x), axis=0))
        return s, ss, m

    z = jnp.zeros((8, 128), jnp.float32)
    s, ss, m = jax.lax.fori_loop(0, _NCHUNK, p1, (z, z, z))

    n = jnp.float32(_N_TOTAL)
    total_s = jnp.sum(s)
    total_ss = jnp.sum(ss)
    max_abs = jnp.max(m)
    var = (total_ss - total_s * total_s / n) / (n - 1.0)
    inv_sig = 1.0 / (jnp.sqrt(var) + 1e-6)
    k_e = _ENERGY_W / (max_abs * max_abs + 1e-6)

    def p2(g, _):
        x = x_vmem[pl.ds(g * _CH, _CH), :]
        e = jnp.exp(2.0 - jnp.abs(x) * inv_sig)
        x_vmem[pl.ds(g * _CH, _CH), :] = (
            _SNR_W / (1.0 + e) + k_e * (x * x) + _RECENCY_C
        )
        pltpu.make_async_copy(
            x_vmem.at[pl.ds(g * _CH, _CH)],
            o_hbm.at[pl.ds(g * _CH, _CH)],
            sem_out.at[g],
        ).start()
        return 0

    jax.lax.fori_loop(0, _NCHUNK, p2, 0)

    def drain(g, _):
        pltpu.make_async_copy(
            x_vmem.at[pl.ds(g * _CH, _CH)],
            o_hbm.at[pl.ds(g * _CH, _CH)],
            sem_out.at[g],
        ).wait()
        return 0

    jax.lax.fori_loop(0, _NCHUNK, drain, 0)


def kernel(weights):
    x = weights.reshape(_ROWS, 128)
    out = pl.pallas_call(
        _fused_kernel,
        in_specs=[pl.BlockSpec(memory_space=pl.ANY)],
        out_specs=pl.BlockSpec(memory_space=pl.ANY),
        out_shape=jax.ShapeDtypeStruct((_ROWS, 128), jnp.float32),
        scratch_shapes=[
            pltpu.VMEM((_ROWS, 128), jnp.float32),
            pltpu.SemaphoreType.DMA((_NCHUNK,)),
            pltpu.SemaphoreType.DMA((_NCHUNK,)),
        ],
    )(x)
    return out.reshape(weights.shape)
